# Initial kernel scaffold; baseline (speedup 1.0000x reference)
import functools

import jax
import jax.numpy as jnp
from jax import lax
from jax.experimental import pallas as pl
from jax.experimental.pallas import tpu as pltpu
from jax.experimental.pallas import tpu_sc as plsc

NUM_GRAPHS = 64
REUSE_WEIGHT = 0.01

NC = 2
NS = 16
L = 16
NW = NC * NS
ROWS = 100000
D = 128
RPW = ROWS // NW
CHUNK = 125
CHUNK_PAD = 128
NCHUNK = RPW // CHUNK
SEG_PAD = NUM_GRAPHS + 1


def _seg_body(codes_hbm, batch_hbm, sums_out, cnts_out,
              idx_v, buf_v, ones_v, sums_sh, cnts_sh):
    c = lax.axis_index("c")
    s = lax.axis_index("s")
    wid = s * NC + c
    base = wid * RPW

    zvec = jnp.zeros((L,), jnp.float32)

    @pl.when(s == 0)
    def _init():
        def zrow(i, carry):
            for jj in range(D // L):
                buf_v[i, pl.ds(jj * L, L)] = zvec
            ones_v[i, :] = zvec
            return carry
        lax.fori_loop(0, SEG_PAD, zrow, 0)
        pltpu.sync_copy(buf_v.at[pl.ds(0, SEG_PAD)], sums_sh)
        pltpu.sync_copy(ones_v.at[pl.ds(0, SEG_PAD)], cnts_sh)

    plsc.subcore_barrier()

    ovec = jnp.ones((L,), jnp.float32)

    def orow(i, carry):
        ones_v[i, :] = ovec
        return carry
    lax.fori_loop(0, CHUNK_PAD, orow, 0)

    pltpu.sync_copy(batch_hbm.at[pl.ds(wid * NCHUNK, NCHUNK)], idx_v)

    def body(j, carry):
        pltpu.sync_copy(codes_hbm.at[pl.ds(base + j * CHUNK, CHUNK)],
                        buf_v.at[pl.ds(0, CHUNK)])
        pltpu.sync_copy(buf_v, sums_sh.at[idx_v.at[j]], add=True)
        pltpu.sync_copy(ones_v, cnts_sh.at[idx_v.at[j]], add=True)
        return carry
    lax.fori_loop(0, NCHUNK, body, 0)

    plsc.subcore_barrier()

    @pl.when(s == 0)
    def _flush():
        pltpu.sync_copy(sums_sh, buf_v.at[pl.ds(0, SEG_PAD)])
        pltpu.sync_copy(buf_v.at[pl.ds(0, SEG_PAD)], sums_out.at[c])
        pltpu.sync_copy(cnts_sh, ones_v.at[pl.ds(0, SEG_PAD)])
        pltpu.sync_copy(ones_v.at[pl.ds(0, SEG_PAD)], cnts_out.at[c])


_seg_reduce = functools.partial(
    pl.kernel,
    out_type=[
        jax.ShapeDtypeStruct((NC, SEG_PAD, D), jnp.float32),
        jax.ShapeDtypeStruct((NC, SEG_PAD, L), jnp.float32),
    ],
    mesh=plsc.VectorSubcoreMesh(core_axis_name="c", subcore_axis_name="s"),
    scratch_types=[
        pltpu.VMEM((NCHUNK, CHUNK_PAD), jnp.int32),
        pltpu.VMEM((CHUNK_PAD, D), jnp.float32),
        pltpu.VMEM((CHUNK_PAD, L), jnp.float32),
        pltpu.VMEM_SHARED((SEG_PAD, D), jnp.float32),
        pltpu.VMEM_SHARED((SEG_PAD, L), jnp.float32),
    ],
)(_seg_body)


def _fin_body(s_ref, c_ref, o_ref):
    sums = s_ref[0, :NUM_GRAPHS, :] + s_ref[1, :NUM_GRAPHS, :]
    counts = c_ref[0, :NUM_GRAPHS, 0:1] + c_ref[1, :NUM_GRAPHS, 0:1]
    means = sums / counts
    mu = jnp.mean(means, axis=0, keepdims=True)
    dev = means - mu
    var = jnp.sum(dev * dev, axis=0) / (NUM_GRAPHS - 1)
    o_ref[...] = jnp.reshape(-REUSE_WEIGHT * jnp.mean(var), (1, 1))


def kernel(sparse_codes, batch):
    batch2d = jnp.pad(
        batch.astype(jnp.int32).reshape(NW * NCHUNK, CHUNK),
        ((0, 0), (0, CHUNK_PAD - CHUNK)),
        constant_values=NUM_GRAPHS,
    )
    sums, cnts = _seg_reduce(sparse_codes, batch2d)
    out = pl.pallas_call(
        _fin_body,
        out_shape=jax.ShapeDtypeStruct((1, 1), jnp.float32),
    )(sums, cnts)
    return out[0, 0]


# speedup vs baseline: 4.9718x; 4.9718x over previous
import functools

import jax
import jax.numpy as jnp
from jax import lax
from jax.experimental import pallas as pl
from jax.experimental.pallas import tpu as pltpu
from jax.experimental.pallas import tpu_sc as plsc

NUM_GRAPHS = 64
REUSE_WEIGHT = 0.01

NC = 2
NS = 16
L = 16
NW = NC * NS
ROWS = 100000
D = 128
RPW = ROWS // NW
CHUNK = 125
CHUNK_PAD = 128
NCHUNK = RPW // CHUNK
SEG_PAD = NUM_GRAPHS + 1


def _seg_body(codes_hbm, batch_hbm, sums_out, cnts_out,
              idx_v, buf_v, ones_v, sums_sh, cnts_sh):
    c = lax.axis_index("c")
    s = lax.axis_index("s")
    wid = s * NC + c
    base = wid * RPW

    zvec = jnp.zeros((L,), jnp.float32)

    @pl.when(s == 0)
    def _init():
        def zrow(i, carry):
            for jj in range(D // L):
                buf_v[i, pl.ds(jj * L, L)] = zvec
            ones_v[i, :] = zvec
            return carry
        lax.fori_loop(0, SEG_PAD, zrow, 0)
        pltpu.sync_copy(buf_v.at[pl.ds(0, SEG_PAD)], sums_sh)
        pltpu.sync_copy(ones_v.at[pl.ds(0, SEG_PAD)], cnts_sh)

    plsc.subcore_barrier()

    ovec = jnp.ones((L,), jnp.float32)

    def orow(i, carry):
        ones_v[i, :] = ovec
        return carry
    lax.fori_loop(0, CHUNK_PAD, orow, 0)

    pltpu.sync_copy(batch_hbm.at[pl.ds(wid * NCHUNK, NCHUNK)], idx_v)

    def body(j, carry):
        pltpu.sync_copy(codes_hbm.at[pl.ds(base + j * CHUNK, CHUNK)],
                        buf_v.at[pl.ds(0, CHUNK)])
        pltpu.sync_copy(buf_v, sums_sh.at[idx_v.at[j]], add=True)
        pltpu.sync_copy(ones_v, cnts_sh.at[idx_v.at[j]], add=True)
        return carry
    lax.fori_loop(0, NCHUNK, body, 0)

    plsc.subcore_barrier()

    @pl.when(s == 0)
    def _flush():
        pltpu.sync_copy(sums_sh, buf_v.at[pl.ds(0, SEG_PAD)])
        pltpu.sync_copy(buf_v.at[pl.ds(0, SEG_PAD)], sums_out.at[c])
        pltpu.sync_copy(cnts_sh, ones_v.at[pl.ds(0, SEG_PAD)])
        pltpu.sync_copy(ones_v.at[pl.ds(0, SEG_PAD)], cnts_out.at[c])


@functools.lru_cache(maxsize=1)
def _make_seg_reduce():
    return functools.partial(
        pl.kernel,
        out_type=[
            jax.ShapeDtypeStruct((NC, SEG_PAD, D), jnp.float32),
            jax.ShapeDtypeStruct((NC, SEG_PAD, L), jnp.float32),
        ],
        mesh=plsc.VectorSubcoreMesh(core_axis_name="c", subcore_axis_name="s"),
        scratch_types=[
            pltpu.VMEM((NCHUNK, CHUNK_PAD), jnp.int32),
            pltpu.VMEM((CHUNK_PAD, D), jnp.float32),
            pltpu.VMEM((CHUNK_PAD, L), jnp.float32),
            pltpu.VMEM_SHARED((SEG_PAD, D), jnp.float32),
            pltpu.VMEM_SHARED((SEG_PAD, L), jnp.float32),
        ],
        compiler_params=pltpu.CompilerParams(use_tc_tiling_on_sc=False),
    )(_seg_body)


def _fin_body(s_ref, c_ref, o_ref):
    sums = s_ref[0, :NUM_GRAPHS, :] + s_ref[1, :NUM_GRAPHS, :]
    counts = c_ref[0, :NUM_GRAPHS, 0:1] + c_ref[1, :NUM_GRAPHS, 0:1]
    means = sums / counts
    mu = jnp.mean(means, axis=0, keepdims=True)
    dev = means - mu
    var = jnp.sum(dev * dev, axis=0) / (NUM_GRAPHS - 1)
    o_ref[...] = jnp.reshape(-REUSE_WEIGHT * jnp.mean(var), (1, 1))


def kernel(sparse_codes, batch):
    batch2d = jnp.pad(
        batch.astype(jnp.int32).reshape(NW * NCHUNK, CHUNK),
        ((0, 0), (0, CHUNK_PAD - CHUNK)),
        constant_values=NUM_GRAPHS,
    )
    sums, cnts = _make_seg_reduce()(sparse_codes, batch2d)
    out = pl.pallas_call(
        _fin_body,
        out_shape=jax.ShapeDtypeStruct((1, 1), jnp.float32),
    )(sums, cnts)
    return out[0, 0]


# speedup vs baseline: 5.7545x; 1.1574x over previous
import functools

import jax
import jax.numpy as jnp
from jax import lax
from jax.experimental import pallas as pl
from jax.experimental.pallas import tpu as pltpu
from jax.experimental.pallas import tpu_sc as plsc

NUM_GRAPHS = 64
REUSE_WEIGHT = 0.01

NC = 2
NS = 16
L = 16
NW = NC * NS
ROWS = 100000
D = 128
RPW = ROWS // NW
CHUNK = 125
CHUNK_PAD = 128
NCHUNK = RPW // CHUNK
SEG_PAD = NUM_GRAPHS + 1


def _seg_body(codes_hbm, batch_hbm, sums_out, cnts_out,
              idx_v, buf_a, buf_b, ones_v, sums_sh, cnts_sh, sem_a, sem_b):
    c = lax.axis_index("c")
    s = lax.axis_index("s")
    wid = s * NC + c
    base = wid * RPW

    zvec = jnp.zeros((L,), jnp.float32)

    @pl.when(s == 0)
    def _init():
        def zrow(i, carry):
            for jj in range(D // L):
                buf_a[i, pl.ds(jj * L, L)] = zvec
            ones_v[i, :] = zvec
            return carry
        lax.fori_loop(0, SEG_PAD, zrow, 0)
        pltpu.sync_copy(buf_a.at[pl.ds(0, SEG_PAD)], sums_sh)
        pltpu.sync_copy(ones_v.at[pl.ds(0, SEG_PAD)], cnts_sh)

    plsc.subcore_barrier()

    ovec = jnp.ones((L,), jnp.float32)

    def orow(i, carry):
        ones_v[i, :] = ovec
        return carry
    lax.fori_loop(0, CHUNK_PAD, orow, 0)

    def load(j, buf, sem):
        return pltpu.async_copy(codes_hbm.at[pl.ds(base + j * CHUNK, CHUNK)],
                                buf.at[pl.ds(0, CHUNK)], sem)

    def wait(j, buf, sem):
        pltpu.make_async_copy(codes_hbm.at[pl.ds(base + j * CHUNK, CHUNK)],
                              buf.at[pl.ds(0, CHUNK)], sem).wait()

    def scatter(j, buf):
        pltpu.sync_copy(buf, sums_sh.at[idx_v.at[j]], add=True)
        pltpu.sync_copy(ones_v, cnts_sh.at[idx_v.at[j]], add=True)

    load(0, buf_a, sem_a)
    pltpu.sync_copy(batch_hbm.at[pl.ds(wid * NCHUNK, NCHUNK)], idx_v)

    def pair(m, carry):
        a = 2 * m
        wait(a, buf_a, sem_a)
        load(a + 1, buf_b, sem_b)
        scatter(a, buf_a)
        wait(a + 1, buf_b, sem_b)
        load(a + 2, buf_a, sem_a)
        scatter(a + 1, buf_b)
        return carry
    lax.fori_loop(0, (NCHUNK - 1) // 2, pair, 0)

    wait(NCHUNK - 1, buf_a, sem_a)
    scatter(NCHUNK - 1, buf_a)

    plsc.subcore_barrier()

    @pl.when(s == 0)
    def _flush():
        pltpu.sync_copy(sums_sh, buf_a.at[pl.ds(0, SEG_PAD)])
        pltpu.sync_copy(buf_a.at[pl.ds(0, SEG_PAD)], sums_out.at[c])
        pltpu.sync_copy(cnts_sh, ones_v.at[pl.ds(0, SEG_PAD)])
        pltpu.sync_copy(ones_v.at[pl.ds(0, SEG_PAD)], cnts_out.at[c])


@functools.lru_cache(maxsize=1)
def _make_seg_reduce():
    return functools.partial(
        pl.kernel,
        out_type=[
            jax.ShapeDtypeStruct((NC, SEG_PAD, D), jnp.float32),
            jax.ShapeDtypeStruct((NC, SEG_PAD, L), jnp.float32),
        ],
        mesh=plsc.VectorSubcoreMesh(core_axis_name="c", subcore_axis_name="s"),
        scratch_types=[
            pltpu.VMEM((NCHUNK, CHUNK_PAD), jnp.int32),
            pltpu.VMEM((CHUNK_PAD, D), jnp.float32),
            pltpu.VMEM((CHUNK_PAD, D), jnp.float32),
            pltpu.VMEM((CHUNK_PAD, L), jnp.float32),
            pltpu.VMEM_SHARED((SEG_PAD, D), jnp.float32),
            pltpu.VMEM_SHARED((SEG_PAD, L), jnp.float32),
            pltpu.SemaphoreType.DMA,
            pltpu.SemaphoreType.DMA,
        ],
        compiler_params=pltpu.CompilerParams(use_tc_tiling_on_sc=False),
    )(_seg_body)


def _fin_body(s_ref, c_ref, o_ref):
    sums = s_ref[0, :NUM_GRAPHS, :] + s_ref[1, :NUM_GRAPHS, :]
    counts = c_ref[0, :NUM_GRAPHS, 0:1] + c_ref[1, :NUM_GRAPHS, 0:1]
    means = sums / counts
    mu = jnp.mean(means, axis=0, keepdims=True)
    dev = means - mu
    var = jnp.sum(dev * dev, axis=0) / (NUM_GRAPHS - 1)
    o_ref[...] = jnp.reshape(-REUSE_WEIGHT * jnp.mean(var), (1, 1))


def kernel(sparse_codes, batch):
    batch2d = jnp.pad(
        batch.astype(jnp.int32).reshape(NW * NCHUNK, CHUNK),
        ((0, 0), (0, CHUNK_PAD - CHUNK)),
        constant_values=NUM_GRAPHS,
    )
    sums, cnts = _make_seg_reduce()(sparse_codes, batch2d)
    out = pl.pallas_call(
        _fin_body,
        out_shape=jax.ShapeDtypeStruct((1, 1), jnp.float32),
    )(sums, cnts)
    return out[0, 0]
